# baseline (speedup 1.0000x reference)
import functools
import numpy as np
import jax
import jax.numpy as jnp
from jax import lax
from jax.experimental import pallas as pl
from jax.experimental.pallas import tpu as pltpu
from jax.experimental.pallas import tpu_sc as plsc

_B, _S = 1024, 200
_SP = 256
_BB = 256
_UNROLL = 8
_NC, _NS, _L = 2, 16, 16
_RPW = _B // (_NC * _NS)

_LOG2 = float(np.log(2))


def _decode_kernel(W_ref, b_ref, T_ref, C_ref, lp_ref, rk_ref, av_ref):
    T = T_ref[...]
    C = C_ref[...]
    D = T

    def q16(x):
        return x.astype(jnp.bfloat16).astype(jnp.float32)

    feats = [C / T, D / T, (T - C) / 1000.0, (T - D) / 1000.0,
             jnp.log(T) / _LOG2, jnp.log(C / _LOG2), jnp.log(D / _LOG2),
             T / 1000.0, C / 1000.0, D / 1000.0]
    p = [q16(feats[k]) * W_ref[k] for k in range(10)]
    while len(p) > 1:
        p = [p[i] + p[i + 1] if i + 1 < len(p) else p[i]
             for i in range(0, len(p), 2)]
    s = p[0] + b_ref[0]

    lane = jax.lax.broadcasted_iota(jnp.int32, (_BB, _SP), 1)
    valid = lane < _S
    s = jnp.where(valid, s, -jnp.inf)

    def pair_step(k, carry):
        rank, q = carry
        for u in range(_UNROLL):
            d = k * _UNROLL + u + 1
            rolled = pltpu.roll(s, d, axis=1)
            diff = rolled - s
            jlt = lane >= d
            beats = (rolled > s) | ((rolled >= s) & jlt)
            rank = rank + jnp.where(beats, 1.0, 0.0)
            rank = rank + pltpu.roll(
                jnp.where(beats, 0.0, 1.0), _SP - d, axis=1)
            e = jnp.exp(jnp.where(beats, -diff, diff))
            q = q + jnp.where(beats, 0.0, e)
            q = q + pltpu.roll(jnp.where(beats, e, 0.0), _SP - d, axis=1)
        return rank, q

    rank0 = jnp.zeros((_BB, _SP), jnp.float32)
    q0 = jnp.zeros((_BB, _SP), jnp.float32)
    rankf, q = jax.lax.fori_loop(
        0, _SP // 2 // _UNROLL, pair_step, (rank0, q0), unroll=False)

    dh = _SP // 2
    rolled = pltpu.roll(s, dh, axis=1)
    diff = rolled - s
    jlt = lane >= dh
    beats = (rolled > s) | ((rolled >= s) & jlt)
    rankf = rankf - pltpu.roll(jnp.where(beats, 0.0, 1.0), dh, axis=1)
    e = jnp.exp(jnp.where(beats, -diff, diff))
    q = q - pltpu.roll(jnp.where(beats, e, 0.0), dh, axis=1)
    rank = rankf.astype(jnp.int32)

    logp = -jnp.log(1.0 + q)

    finite = valid & (s > -1e8)
    nf = jnp.sum(jnp.where(finite, 1, 0), axis=1, keepdims=True)
    mfi = jnp.min(jnp.where(finite, lane, _SP * 4), axis=1, keepdims=True)
    degen = s < -1e8
    logp = jnp.where(degen, -jnp.log(nf.astype(jnp.float32)), logp)
    act_val = jnp.where(degen, mfi, lane)

    lp_ref[...] = logp
    rk_ref[...] = rank
    av_ref[...] = act_val


def _scatter_body(lp_hbm, rk_hbm, av_hbm, probs_hbm, acts_hbm,
                  lp_v, rk_v, av_v, op_v, oa_v):
    wid = lax.axis_index("s") * _NC + lax.axis_index("c")
    base = wid * _RPW
    pltpu.sync_copy(lp_hbm.at[pl.ds(base, _RPW)], lp_v)
    pltpu.sync_copy(rk_hbm.at[pl.ds(base, _RPW)], rk_v)
    pltpu.sync_copy(av_hbm.at[pl.ds(base, _RPW)], av_v)

    def row(r, carry):
        rr = jnp.full((_L,), r, jnp.int32)
        for ch in range(_SP // _L):
            sl = pl.ds(ch * _L, _L)
            idx = rk_v[r, sl]
            plsc.store_scatter(op_v, [rr, idx], lp_v[r, sl])
            plsc.store_scatter(oa_v, [rr, idx], av_v[r, sl])
        return carry
    lax.fori_loop(0, _RPW, row, 0)

    pltpu.sync_copy(op_v, probs_hbm.at[pl.ds(base, _RPW)])
    pltpu.sync_copy(oa_v, acts_hbm.at[pl.ds(base, _RPW)])


def _bf16_rne(w):
    v = jax.lax.bitcast_convert_type(w, jnp.int32)
    r = v + 0x7FFF + ((v >> 16) & 1)
    r = jnp.where(jnp.isnan(w) | jnp.isinf(w), v, r)
    return jax.lax.bitcast_convert_type(
        r & jnp.int32(-65536), jnp.float32)


def kernel(inputs, argmax, get_reward, W, b):
    T0 = inputs[:, :, 0]
    C0 = inputs[:, :, 1]
    Wq = _bf16_rne(W)
    Tp = jnp.pad(T0, ((0, 0), (0, _SP - _S)), constant_values=1.0)
    Cp = jnp.pad(C0, ((0, 0), (0, _SP - _S)), constant_values=1.0)

    lp, rk, av = pl.pallas_call(
        _decode_kernel,
        grid=(_B // _BB,),
        in_specs=[
            pl.BlockSpec(memory_space=pltpu.SMEM),
            pl.BlockSpec(memory_space=pltpu.SMEM),
            pl.BlockSpec((_BB, _SP), lambda g: (g, 0)),
            pl.BlockSpec((_BB, _SP), lambda g: (g, 0)),
        ],
        out_specs=(
            pl.BlockSpec((_BB, _SP), lambda g: (g, 0)),
            pl.BlockSpec((_BB, _SP), lambda g: (g, 0)),
            pl.BlockSpec((_BB, _SP), lambda g: (g, 0)),
        ),
        out_shape=(
            jax.ShapeDtypeStruct((_B, _SP), jnp.float32),
            jax.ShapeDtypeStruct((_B, _SP), jnp.int32),
            jax.ShapeDtypeStruct((_B, _SP), jnp.int32),
        ),
        compiler_params=pltpu.CompilerParams(
            dimension_semantics=("parallel",),
        ),
    )(Wq, b, Tp, Cp)

    mesh = plsc.VectorSubcoreMesh(
        core_axis_name="c", subcore_axis_name="s",
        num_cores=_NC, num_subcores=_NS)
    scatter = functools.partial(
        pl.kernel, mesh=mesh,
        compiler_params=pltpu.CompilerParams(
            use_tc_tiling_on_sc=False, needs_layout_passes=False),
        out_type=(
            jax.ShapeDtypeStruct((_B, _SP), jnp.float32),
            jax.ShapeDtypeStruct((_B, _SP), jnp.int32),
        ),
        scratch_types=[
            pltpu.VMEM((_RPW, _SP), jnp.float32),
            pltpu.VMEM((_RPW, _SP), jnp.int32),
            pltpu.VMEM((_RPW, _SP), jnp.int32),
            pltpu.VMEM((_RPW, _SP), jnp.float32),
            pltpu.VMEM((_RPW, _SP), jnp.int32),
        ],
    )(_scatter_body)
    probs, acts = scatter(lp, rk, av)

    return probs[:, :_S], acts[:, :_S]


# speedup vs baseline: 1.3592x; 1.3592x over previous
import functools
import numpy as np
import jax
import jax.numpy as jnp
from jax import lax
from jax.experimental import pallas as pl
from jax.experimental.pallas import tpu as pltpu
from jax.experimental.pallas import tpu_sc as plsc

_B, _S = 1024, 200
_SP = 256
_BB = 256
_UNROLL = 16
_NC, _NS, _L = 2, 16, 16
_RPW = _B // (_NC * _NS)

_LOG2 = float(np.log(2))


def _decode_kernel(W_ref, b_ref, T_ref, C_ref, lp_ref, rk_ref, av_ref):
    T = T_ref[...]
    C = C_ref[...]
    D = T

    def q16(x):
        return x.astype(jnp.bfloat16).astype(jnp.float32)

    feats = [C / T, D / T, (T - C) / 1000.0, (T - D) / 1000.0,
             jnp.log(T) / _LOG2, jnp.log(C / _LOG2), jnp.log(D / _LOG2),
             T / 1000.0, C / 1000.0, D / 1000.0]
    p = [q16(feats[k]) * W_ref[k] for k in range(10)]
    while len(p) > 1:
        p = [p[i] + p[i + 1] if i + 1 < len(p) else p[i]
             for i in range(0, len(p), 2)]
    s = p[0] + b_ref[0]

    lane = jax.lax.broadcasted_iota(jnp.int32, (_BB, _SP), 1)
    valid = lane < _S
    s = jnp.where(valid, s, -jnp.inf)

    def pair_step(k, carry):
        rank, q = carry
        for u in range(_UNROLL):
            d = k * _UNROLL + u
            rolled = pltpu.roll(s, d, axis=1)
            jlt = lane >= d
            beats = (rolled > s) | ((rolled == s) & jlt)
            rank = rank + beats.astype(jnp.int32)
            q = q + jnp.where(beats, 0.0, jnp.exp(rolled - s))
        return rank, q

    rank0 = jnp.full((_BB, _SP), -1, jnp.int32)
    q0 = jnp.zeros((_BB, _SP), jnp.float32)
    rank, q = jax.lax.fori_loop(
        0, _SP // _UNROLL, pair_step, (rank0, q0), unroll=False)

    logp = -jnp.log(1.0 + q)

    finite = valid & (s > -1e8)
    nf = jnp.sum(jnp.where(finite, 1, 0), axis=1, keepdims=True)
    mfi = jnp.min(jnp.where(finite, lane, _SP * 4), axis=1, keepdims=True)
    degen = s < -1e8
    logp = jnp.where(degen, -jnp.log(nf.astype(jnp.float32)), logp)
    act_val = jnp.where(degen, mfi, lane)

    lp_ref[...] = logp
    rk_ref[...] = rank
    av_ref[...] = act_val


def _scatter_body(lp_hbm, rk_hbm, av_hbm, probs_hbm, acts_hbm,
                  lp_v, rk_v, av_v, op_v, oa_v):
    wid = lax.axis_index("s") * _NC + lax.axis_index("c")
    base = wid * _RPW
    pltpu.sync_copy(lp_hbm.at[pl.ds(base, _RPW)], lp_v)
    pltpu.sync_copy(rk_hbm.at[pl.ds(base, _RPW)], rk_v)
    pltpu.sync_copy(av_hbm.at[pl.ds(base, _RPW)], av_v)

    def row(r, carry):
        rr = jnp.full((_L,), r, jnp.int32)
        for ch in range(_SP // _L):
            sl = pl.ds(ch * _L, _L)
            idx = rk_v[r, sl]
            plsc.store_scatter(op_v, [rr, idx], lp_v[r, sl])
            plsc.store_scatter(oa_v, [rr, idx], av_v[r, sl])
        return carry
    lax.fori_loop(0, _RPW, row, 0)

    pltpu.sync_copy(op_v, probs_hbm.at[pl.ds(base, _RPW)])
    pltpu.sync_copy(oa_v, acts_hbm.at[pl.ds(base, _RPW)])


def _bf16_rne(w):
    v = jax.lax.bitcast_convert_type(w, jnp.int32)
    r = v + 0x7FFF + ((v >> 16) & 1)
    r = jnp.where(jnp.isnan(w) | jnp.isinf(w), v, r)
    return jax.lax.bitcast_convert_type(
        r & jnp.int32(-65536), jnp.float32)


def kernel(inputs, argmax, get_reward, W, b):
    T0 = inputs[:, :, 0]
    C0 = inputs[:, :, 1]
    Wq = _bf16_rne(W)
    Tp = jnp.pad(T0, ((0, 0), (0, _SP - _S)), constant_values=1.0)
    Cp = jnp.pad(C0, ((0, 0), (0, _SP - _S)), constant_values=1.0)

    lp, rk, av = pl.pallas_call(
        _decode_kernel,
        grid=(_B // _BB,),
        in_specs=[
            pl.BlockSpec(memory_space=pltpu.SMEM),
            pl.BlockSpec(memory_space=pltpu.SMEM),
            pl.BlockSpec((_BB, _SP), lambda g: (g, 0)),
            pl.BlockSpec((_BB, _SP), lambda g: (g, 0)),
        ],
        out_specs=(
            pl.BlockSpec((_BB, _SP), lambda g: (g, 0)),
            pl.BlockSpec((_BB, _SP), lambda g: (g, 0)),
            pl.BlockSpec((_BB, _SP), lambda g: (g, 0)),
        ),
        out_shape=(
            jax.ShapeDtypeStruct((_B, _SP), jnp.float32),
            jax.ShapeDtypeStruct((_B, _SP), jnp.int32),
            jax.ShapeDtypeStruct((_B, _SP), jnp.int32),
        ),
        compiler_params=pltpu.CompilerParams(
            dimension_semantics=("parallel",),
        ),
    )(Wq, b, Tp, Cp)

    mesh = plsc.VectorSubcoreMesh(
        core_axis_name="c", subcore_axis_name="s",
        num_cores=_NC, num_subcores=_NS)
    scatter = functools.partial(
        pl.kernel, mesh=mesh,
        compiler_params=pltpu.CompilerParams(
            use_tc_tiling_on_sc=False, needs_layout_passes=False),
        out_type=(
            jax.ShapeDtypeStruct((_B, _SP), jnp.float32),
            jax.ShapeDtypeStruct((_B, _SP), jnp.int32),
        ),
        scratch_types=[
            pltpu.VMEM((_RPW, _SP), jnp.float32),
            pltpu.VMEM((_RPW, _SP), jnp.int32),
            pltpu.VMEM((_RPW, _SP), jnp.int32),
            pltpu.VMEM((_RPW, _SP), jnp.float32),
            pltpu.VMEM((_RPW, _SP), jnp.int32),
        ],
    )(_scatter_body)
    probs, acts = scatter(lp, rk, av)

    return probs[:, :_S], acts[:, :_S]


# speedup vs baseline: 1.4132x; 1.0397x over previous
import functools
import numpy as np
import jax
import jax.numpy as jnp
from jax import lax
from jax.experimental import pallas as pl
from jax.experimental.pallas import tpu as pltpu
from jax.experimental.pallas import tpu_sc as plsc

_B, _S = 1024, 200
_SP = 256
_BB = 256
_UNROLL = 32
_NC, _NS, _L = 2, 16, 16
_RPW = _B // (_NC * _NS)

_LOG2 = float(np.log(2))


def _decode_kernel(W_ref, b_ref, T_ref, C_ref, lp_ref, rk_ref, av_ref):
    T = T_ref[...]
    C = C_ref[...]
    D = T

    def q16(x):
        return x.astype(jnp.bfloat16).astype(jnp.float32)

    feats = [C / T, D / T, (T - C) / 1000.0, (T - D) / 1000.0,
             jnp.log(T) / _LOG2, jnp.log(C / _LOG2), jnp.log(D / _LOG2),
             T / 1000.0, C / 1000.0, D / 1000.0]
    p = [q16(feats[k]) * W_ref[k] for k in range(10)]
    while len(p) > 1:
        p = [p[i] + p[i + 1] if i + 1 < len(p) else p[i]
             for i in range(0, len(p), 2)]
    s = p[0] + b_ref[0]

    lane = jax.lax.broadcasted_iota(jnp.int32, (_BB, _SP), 1)
    valid = lane < _S
    s = jnp.where(valid, s, -jnp.inf)

    def pair_step(k, carry):
        rank, q = carry
        for u in range(_UNROLL):
            d = k * _UNROLL + u
            rolled = pltpu.roll(s, d, axis=1)
            jlt = lane >= d
            beats = (rolled > s) | ((rolled == s) & jlt)
            rank = rank + beats.astype(jnp.int32)
            q = q + jnp.where(beats, 0.0, jnp.exp(rolled - s))
        return rank, q

    rank0 = jnp.full((_BB, _SP), -1, jnp.int32)
    q0 = jnp.zeros((_BB, _SP), jnp.float32)
    rank, q = jax.lax.fori_loop(
        0, _SP // _UNROLL, pair_step, (rank0, q0), unroll=False)

    logp = -jnp.log(1.0 + q)

    finite = valid & (s > -1e8)
    nf = jnp.sum(jnp.where(finite, 1, 0), axis=1, keepdims=True)
    mfi = jnp.min(jnp.where(finite, lane, _SP * 4), axis=1, keepdims=True)
    degen = s < -1e8
    logp = jnp.where(degen, -jnp.log(nf.astype(jnp.float32)), logp)
    act_val = jnp.where(degen, mfi, lane)

    lp_ref[...] = logp
    rk_ref[...] = rank
    av_ref[...] = act_val


def _scatter_body(lp_hbm, rk_hbm, av_hbm, probs_hbm, acts_hbm,
                  lp_v, rk_v, av_v, op_v, oa_v):
    wid = lax.axis_index("s") * _NC + lax.axis_index("c")
    base = wid * _RPW
    pltpu.sync_copy(lp_hbm.at[pl.ds(base, _RPW)], lp_v)
    pltpu.sync_copy(rk_hbm.at[pl.ds(base, _RPW)], rk_v)
    pltpu.sync_copy(av_hbm.at[pl.ds(base, _RPW)], av_v)

    def row(r, carry):
        rr = jnp.full((_L,), r, jnp.int32)
        for ch in range(_SP // _L):
            sl = pl.ds(ch * _L, _L)
            idx = rk_v[r, sl]
            plsc.store_scatter(op_v, [rr, idx], lp_v[r, sl])
            plsc.store_scatter(oa_v, [rr, idx], av_v[r, sl])
        return carry
    lax.fori_loop(0, _RPW, row, 0)

    pltpu.sync_copy(op_v, probs_hbm.at[pl.ds(base, _RPW)])
    pltpu.sync_copy(oa_v, acts_hbm.at[pl.ds(base, _RPW)])


def _bf16_rne(w):
    v = jax.lax.bitcast_convert_type(w, jnp.int32)
    r = v + 0x7FFF + ((v >> 16) & 1)
    r = jnp.where(jnp.isnan(w) | jnp.isinf(w), v, r)
    return jax.lax.bitcast_convert_type(
        r & jnp.int32(-65536), jnp.float32)


def kernel(inputs, argmax, get_reward, W, b):
    T0 = inputs[:, :, 0]
    C0 = inputs[:, :, 1]
    Wq = _bf16_rne(W)
    Tp = jnp.pad(T0, ((0, 0), (0, _SP - _S)), constant_values=1.0)
    Cp = jnp.pad(C0, ((0, 0), (0, _SP - _S)), constant_values=1.0)

    lp, rk, av = pl.pallas_call(
        _decode_kernel,
        grid=(_B // _BB,),
        in_specs=[
            pl.BlockSpec(memory_space=pltpu.SMEM),
            pl.BlockSpec(memory_space=pltpu.SMEM),
            pl.BlockSpec((_BB, _SP), lambda g: (g, 0)),
            pl.BlockSpec((_BB, _SP), lambda g: (g, 0)),
        ],
        out_specs=(
            pl.BlockSpec((_BB, _SP), lambda g: (g, 0)),
            pl.BlockSpec((_BB, _SP), lambda g: (g, 0)),
            pl.BlockSpec((_BB, _SP), lambda g: (g, 0)),
        ),
        out_shape=(
            jax.ShapeDtypeStruct((_B, _SP), jnp.float32),
            jax.ShapeDtypeStruct((_B, _SP), jnp.int32),
            jax.ShapeDtypeStruct((_B, _SP), jnp.int32),
        ),
        compiler_params=pltpu.CompilerParams(
            dimension_semantics=("parallel",),
        ),
    )(Wq, b, Tp, Cp)

    mesh = plsc.VectorSubcoreMesh(
        core_axis_name="c", subcore_axis_name="s",
        num_cores=_NC, num_subcores=_NS)
    scatter = functools.partial(
        pl.kernel, mesh=mesh,
        compiler_params=pltpu.CompilerParams(
            use_tc_tiling_on_sc=False, needs_layout_passes=False),
        out_type=(
            jax.ShapeDtypeStruct((_B, _SP), jnp.float32),
            jax.ShapeDtypeStruct((_B, _SP), jnp.int32),
        ),
        scratch_types=[
            pltpu.VMEM((_RPW, _SP), jnp.float32),
            pltpu.VMEM((_RPW, _SP), jnp.int32),
            pltpu.VMEM((_RPW, _SP), jnp.int32),
            pltpu.VMEM((_RPW, _SP), jnp.float32),
            pltpu.VMEM((_RPW, _SP), jnp.int32),
        ],
    )(_scatter_body)
    probs, acts = scatter(lp, rk, av)

    return probs[:, :_S], acts[:, :_S]


# speedup vs baseline: 1.4395x; 1.0186x over previous
import functools
import numpy as np
import jax
import jax.numpy as jnp
from jax import lax
from jax.experimental import pallas as pl
from jax.experimental.pallas import tpu as pltpu
from jax.experimental.pallas import tpu_sc as plsc

_B, _S = 1024, 200
_SP = 256
_BB = 256
_UNROLL = 64
_NC, _NS, _L = 2, 16, 16
_RPW = _B // (_NC * _NS)

_LOG2 = float(np.log(2))


def _decode_kernel(W_ref, b_ref, T_ref, C_ref, lp_ref, rk_ref, av_ref):
    T = T_ref[...]
    C = C_ref[...]
    D = T

    def q16(x):
        return x.astype(jnp.bfloat16).astype(jnp.float32)

    feats = [C / T, D / T, (T - C) / 1000.0, (T - D) / 1000.0,
             jnp.log(T) / _LOG2, jnp.log(C / _LOG2), jnp.log(D / _LOG2),
             T / 1000.0, C / 1000.0, D / 1000.0]
    p = [q16(feats[k]) * W_ref[k] for k in range(10)]
    while len(p) > 1:
        p = [p[i] + p[i + 1] if i + 1 < len(p) else p[i]
             for i in range(0, len(p), 2)]
    s = p[0] + b_ref[0]

    lane = jax.lax.broadcasted_iota(jnp.int32, (_BB, _SP), 1)
    valid = lane < _S
    s = jnp.where(valid, s, -jnp.inf)

    def pair_step(k, carry):
        rank, q = carry
        for u in range(_UNROLL):
            d = k * _UNROLL + u
            rolled = pltpu.roll(s, d, axis=1)
            jlt = lane >= d
            beats = (rolled > s) | ((rolled == s) & jlt)
            rank = rank + beats.astype(jnp.int32)
            q = q + jnp.where(beats, 0.0, jnp.exp(rolled - s))
        return rank, q

    rank0 = jnp.full((_BB, _SP), -1, jnp.int32)
    q0 = jnp.zeros((_BB, _SP), jnp.float32)
    rank, q = jax.lax.fori_loop(
        0, _SP // _UNROLL, pair_step, (rank0, q0), unroll=False)

    logp = -jnp.log(1.0 + q)

    finite = valid & (s > -1e8)
    nf = jnp.sum(jnp.where(finite, 1, 0), axis=1, keepdims=True)
    mfi = jnp.min(jnp.where(finite, lane, _SP * 4), axis=1, keepdims=True)
    degen = s < -1e8
    logp = jnp.where(degen, -jnp.log(nf.astype(jnp.float32)), logp)
    act_val = jnp.where(degen, mfi, lane)

    lp_ref[...] = logp
    rk_ref[...] = rank
    av_ref[...] = act_val


def _scatter_body(lp_hbm, rk_hbm, av_hbm, probs_hbm, acts_hbm,
                  lp_v, rk_v, av_v, op_v, oa_v):
    wid = lax.axis_index("s") * _NC + lax.axis_index("c")
    base = wid * _RPW
    pltpu.sync_copy(lp_hbm.at[pl.ds(base, _RPW)], lp_v)
    pltpu.sync_copy(rk_hbm.at[pl.ds(base, _RPW)], rk_v)
    pltpu.sync_copy(av_hbm.at[pl.ds(base, _RPW)], av_v)

    def row(r, carry):
        rr = jnp.full((_L,), r, jnp.int32)
        for ch in range(_SP // _L):
            sl = pl.ds(ch * _L, _L)
            idx = rk_v[r, sl]
            plsc.store_scatter(op_v, [rr, idx], lp_v[r, sl])
            plsc.store_scatter(oa_v, [rr, idx], av_v[r, sl])
        return carry
    lax.fori_loop(0, _RPW, row, 0)

    pltpu.sync_copy(op_v, probs_hbm.at[pl.ds(base, _RPW)])
    pltpu.sync_copy(oa_v, acts_hbm.at[pl.ds(base, _RPW)])


def _bf16_rne(w):
    v = jax.lax.bitcast_convert_type(w, jnp.int32)
    r = v + 0x7FFF + ((v >> 16) & 1)
    r = jnp.where(jnp.isnan(w) | jnp.isinf(w), v, r)
    return jax.lax.bitcast_convert_type(
        r & jnp.int32(-65536), jnp.float32)


def kernel(inputs, argmax, get_reward, W, b):
    T0 = inputs[:, :, 0]
    C0 = inputs[:, :, 1]
    Wq = _bf16_rne(W)
    Tp = jnp.pad(T0, ((0, 0), (0, _SP - _S)), constant_values=1.0)
    Cp = jnp.pad(C0, ((0, 0), (0, _SP - _S)), constant_values=1.0)

    lp, rk, av = pl.pallas_call(
        _decode_kernel,
        grid=(_B // _BB,),
        in_specs=[
            pl.BlockSpec(memory_space=pltpu.SMEM),
            pl.BlockSpec(memory_space=pltpu.SMEM),
            pl.BlockSpec((_BB, _SP), lambda g: (g, 0)),
            pl.BlockSpec((_BB, _SP), lambda g: (g, 0)),
        ],
        out_specs=(
            pl.BlockSpec((_BB, _SP), lambda g: (g, 0)),
            pl.BlockSpec((_BB, _SP), lambda g: (g, 0)),
            pl.BlockSpec((_BB, _SP), lambda g: (g, 0)),
        ),
        out_shape=(
            jax.ShapeDtypeStruct((_B, _SP), jnp.float32),
            jax.ShapeDtypeStruct((_B, _SP), jnp.int32),
            jax.ShapeDtypeStruct((_B, _SP), jnp.int32),
        ),
        compiler_params=pltpu.CompilerParams(
            dimension_semantics=("parallel",),
        ),
    )(Wq, b, Tp, Cp)

    mesh = plsc.VectorSubcoreMesh(
        core_axis_name="c", subcore_axis_name="s",
        num_cores=_NC, num_subcores=_NS)
    scatter = functools.partial(
        pl.kernel, mesh=mesh,
        compiler_params=pltpu.CompilerParams(
            use_tc_tiling_on_sc=False, needs_layout_passes=False),
        out_type=(
            jax.ShapeDtypeStruct((_B, _SP), jnp.float32),
            jax.ShapeDtypeStruct((_B, _SP), jnp.int32),
        ),
        scratch_types=[
            pltpu.VMEM((_RPW, _SP), jnp.float32),
            pltpu.VMEM((_RPW, _SP), jnp.int32),
            pltpu.VMEM((_RPW, _SP), jnp.int32),
            pltpu.VMEM((_RPW, _SP), jnp.float32),
            pltpu.VMEM((_RPW, _SP), jnp.int32),
        ],
    )(_scatter_body)
    probs, acts = scatter(lp, rk, av)

    return probs[:, :_S], acts[:, :_S]
